# baseline (speedup 1.0000x reference)
import functools
import jax
import jax.numpy as jnp
from jax import lax
from jax.experimental import pallas as pl
from jax.experimental.pallas import tpu as pltpu
from jax.experimental.pallas import tpu_sc as plsc

N = 10000
D = 128
H = 8
C = 16
L = 3
FF = 512
NPAD = 10112
E = 320000
EPAD = 331776
NC, NS = 2, 16
PER_TILE = EPAD // (NC * NS)
CHUNK = 64
NCHUNK = PER_TILE // CHUNK
NCIO = NCHUNK + 2
STRIPE = NPAD // NS

_mm = functools.partial(jnp.dot, precision=lax.Precision.HIGHEST)
_f32 = jnp.float32


def _rowmask():
    return lax.broadcasted_iota(jnp.int32, (NPAD, D), 0) < N


def _headsel():
    r = lax.broadcasted_iota(jnp.int32, (D, 16), 0) // 16
    c = lax.broadcasted_iota(jnp.int32, (D, 16), 1)
    return (r == c).astype(_f32)


def _headexp():
    r = lax.broadcasted_iota(jnp.int32, (16, D), 0)
    c = lax.broadcasted_iota(jnp.int32, (16, D), 1) // 16
    return (r == c).astype(_f32)


def _halfsum():
    r = lax.broadcasted_iota(jnp.int32, (32, 16), 0) % 16
    c = lax.broadcasted_iota(jnp.int32, (32, 16), 1)
    return (r == c).astype(_f32)


def _proj(x, wgat, af_s, af_d, h_o, as_o, ad_o):
    h = _mm(x, wgat)
    S = _headsel()
    h_o[...] = h
    as_o[...] = _mm(h * af_s, S)
    ad_o[...] = _mm(h * af_d, S)


def _tc_embed_a(xp, wemb, bemb, wgat, af_s, af_d,
                x_o, h_o, as_o, ad_o):
    x = jnp.where(_rowmask(), _mm(xp[...], wemb[...]) + bemb[...], 0.0)
    x_o[...] = x
    _proj(x, wgat[...], af_s[...], af_d[...], h_o, as_o, ad_o)


_FFBLK = 632


def _postgat(x, accA, accD, bgat, g1, b1, g2, b2, w1, bf1, w2, bf2, u_ref,
             v_ref):
    mask = _rowmask()
    inv = 1.0 / (_mm(accD, _halfsum()) + 1e-16)
    y = (accA[0] + accA[1]) * _mm(inv, _headexp()) + bgat
    t = x + y
    mu = jnp.sum(t, axis=0, keepdims=True) / N
    var = jnp.sum(t * t, axis=0, keepdims=True) / N - mu * mu
    u_ref[...] = jnp.where(mask, g1 * (t - mu) * lax.rsqrt(var + 1e-5) + b1,
                           0.0)

    def blk(i, carry):
        off = pl.multiple_of(i * _FFBLK, 8)
        ub = u_ref[pl.ds(off, _FFBLK), :]
        fb = jax.nn.relu(_mm(ub, w1) + bf1)
        v_ref[pl.ds(off, _FFBLK), :] = ub + _mm(fb, w2) + bf2
        return carry

    lax.fori_loop(0, NPAD // _FFBLK, blk, 0)
    v = v_ref[...]
    mu2 = jnp.sum(v, axis=0, keepdims=True) / N
    var2 = jnp.sum(v * v, axis=0, keepdims=True) / N - mu2 * mu2
    return jnp.where(mask, g2 * (v - mu2) * lax.rsqrt(var2 + 1e-5) + b2, 0.0)


def _tc_b(x, accA, accD, bgat, g1, b1, g2, b2, w1, bf1, w2, bf2, x_o,
          u_ref, v_ref):
    x_o[...] = _postgat(x[...], accA[...], accD[...], bgat[...], g1[...],
                        b1[...], g2[...], b2[...], w1[...], bf1[...],
                        w2[...], bf2[...], u_ref, v_ref)


def _tc_a(x, wgat, af_s, af_d, h_o, as_o, ad_o):
    _proj(x[...], wgat[...], af_s[...], af_d[...], h_o, as_o, ad_o)


def _bcast_lane(vec, h):
    idx = jnp.full((16, 1), h, jnp.int32)
    dn = lax.GatherDimensionNumbers(offset_dims=(), collapsed_slice_dims=(0,),
                                    start_index_map=(0,))
    return lax.gather(vec, idx, dn, (1,),
                      mode=lax.GatherScatterMode.PROMISE_IN_BOUNDS)


def _sc_edge_body(h_hbm, as_hbm, ad_hbm, idx_hbm, zA_hbm, zD_hbm,
                  accA_o, accD_o,
                  gidx0, gidx1, scidx0, scidx1,
                  hrow0, as0, ad0, w0, msg0,
                  hrow1, as1, ad1, w1, msg1,
                  accA_s, accD_s,
                  semi0, semi1, semh0, sema0, semd0, semh1, sema1, semd1,
                  semA0, semD0, semA1, semD1):
    cid = lax.axis_index("c")
    sid = lax.axis_index("s")
    wid = sid * NC + cid
    base_rows = sid * STRIPE

    mask16 = jnp.where(lax.iota(jnp.int32, 16) < 8, 1.0, 0.0).astype(_f32)
    slots = ((gidx0, scidx0, hrow0, as0, ad0, w0, msg0,
              semi0, semh0, sema0, semd0, semA0, semD0),
             (gidx1, scidx1, hrow1, as1, ad1, w1, msg1,
              semi1, semh1, sema1, semd1, semA1, semD1))

    def issue_idx(ci, s):
        gi, _, _, _, _, _, _, si = slots[s][:8]
        pltpu.async_copy(idx_hbm.at[wid, ci], gi, si)

    def wait_idx(ci, s):
        gi, _, _, _, _, _, _, si = slots[s][:8]
        pltpu.make_async_copy(idx_hbm.at[wid, ci], gi, si).wait()

    def issue_gather(s):
        gi, _, hv, av, dv, _, _, _, sh, sa, sd = slots[s][:11]
        pltpu.async_copy(h_hbm.at[gi.at[0]], hv, sh)
        pltpu.async_copy(as_hbm.at[gi.at[0]], av, sa)
        pltpu.async_copy(ad_hbm.at[gi.at[1]], dv, sd)

    def wait_gather(s):
        gi, _, hv, av, dv, _, _, _, sh, sa, sd = slots[s][:11]
        pltpu.make_async_copy(h_hbm.at[gi.at[0]], hv, sh).wait()
        pltpu.make_async_copy(as_hbm.at[gi.at[0]], av, sa).wait()
        pltpu.make_async_copy(ad_hbm.at[gi.at[1]], dv, sd).wait()

    def save_scatter_idx(s):
        gi, sci = slots[s][:2]
        for j in range(CHUNK // 16):
            sci[0, pl.ds(j * 16, 16)] = gi[1, pl.ds(j * 16, 16)]

    def compute(s):
        hv, av, dv, wv, mv = slots[s][2:7]

        def edge(e, carry):
            a = av[e, :] + dv[e, :]
            w = jnp.exp(jnp.maximum(a, 0.2 * a)) * mask16
            wv[e, :] = w
            for h in range(H):
                mv[e, pl.ds(h * 16, 16)] = (hv[e, pl.ds(h * 16, 16)]
                                            * _bcast_lane(w, h))
            return carry

        lax.fori_loop(0, CHUNK, edge, 0, unroll=2)

    def issue_scatter(s):
        sci, wv, mv, sA, sD = (slots[s][1], slots[s][5], slots[s][6],
                               slots[s][11], slots[s][12])
        pltpu.async_copy(mv, accA_s.at[sci.at[0]], sA, add=True)
        pltpu.async_copy(wv, accD_s.at[sci.at[0]], sD, add=True)

    def wait_scatter(s):
        sci, wv, mv, sA, sD = (slots[s][1], slots[s][5], slots[s][6],
                               slots[s][11], slots[s][12])
        pltpu.make_async_copy(mv, accA_s.at[sci.at[0]], sA).wait()
        pltpu.make_async_copy(wv, accD_s.at[sci.at[0]], sD).wait()

    pltpu.sync_copy(idx_hbm.at[wid, 0], gidx0)
    pltpu.sync_copy(idx_hbm.at[wid, 1], gidx1)
    issue_gather(0)
    issue_gather(1)
    pltpu.sync_copy(zA_hbm.at[pl.ds(base_rows, STRIPE)],
                    accA_s.at[pl.ds(base_rows, STRIPE)])
    pltpu.sync_copy(zD_hbm.at[pl.ds(base_rows, STRIPE)],
                    accD_s.at[pl.ds(base_rows, STRIPE)])
    plsc.subcore_barrier()

    def first(ci, s):
        wait_gather(s)
        save_scatter_idx(s)
        issue_idx(ci + 2, s)
        compute(s)
        issue_scatter(s)
        wait_idx(ci + 2, s)
        issue_gather(s)

    def one(ci, s):
        wait_scatter(s)
        wait_gather(s)
        save_scatter_idx(s)
        issue_idx(ci + 2, s)
        compute(s)
        issue_scatter(s)
        wait_idx(ci + 2, s)
        issue_gather(s)

    first(0, 0)
    first(1, 1)

    def pair(k, carry):
        one(2 * k, 0)
        one(2 * k + 1, 1)
        return carry

    lax.fori_loop(1, NCHUNK // 2, pair, 0)

    wait_gather(0)
    wait_gather(1)
    wait_scatter(0)
    wait_scatter(1)

    plsc.subcore_barrier()
    pltpu.sync_copy(accA_s.at[pl.ds(base_rows, STRIPE)],
                    accA_o.at[cid, pl.ds(base_rows, STRIPE)])
    pltpu.sync_copy(accD_s.at[pl.ds(base_rows, STRIPE)],
                    accD_o.at[pl.ds(base_rows, STRIPE), pl.ds(cid * 16, 16)])


_sc_edge = pl.kernel(
    _sc_edge_body,
    out_type=(jax.ShapeDtypeStruct((NC, NPAD, D), _f32),
              jax.ShapeDtypeStruct((NPAD, 32), _f32)),
    mesh=plsc.VectorSubcoreMesh(core_axis_name="c", subcore_axis_name="s"),
    compiler_params=pltpu.CompilerParams(use_tc_tiling_on_sc=False),
    scratch_types=(
        pltpu.VMEM((2, CHUNK), jnp.int32),
        pltpu.VMEM((2, CHUNK), jnp.int32),
        pltpu.VMEM((1, CHUNK), jnp.int32),
        pltpu.VMEM((1, CHUNK), jnp.int32),
        pltpu.VMEM((CHUNK, D), _f32),
        pltpu.VMEM((CHUNK, 16), _f32),
        pltpu.VMEM((CHUNK, 16), _f32),
        pltpu.VMEM((CHUNK, 16), _f32),
        pltpu.VMEM((CHUNK, D), _f32),
        pltpu.VMEM((CHUNK, D), _f32),
        pltpu.VMEM((CHUNK, 16), _f32),
        pltpu.VMEM((CHUNK, 16), _f32),
        pltpu.VMEM((CHUNK, 16), _f32),
        pltpu.VMEM((CHUNK, D), _f32),
        pltpu.VMEM_SHARED((NPAD, D), _f32),
        pltpu.VMEM_SHARED((NPAD, 16), _f32),
    ) + (pltpu.SemaphoreType.DMA,) * 12,
)


def _tc_call(body, n_out_like, scratch=False):
    return pl.pallas_call(
        body, out_shape=n_out_like,
        scratch_shapes=[pltpu.VMEM((NPAD, D), _f32),
                        pltpu.VMEM((NPAD, D), _f32)] if scratch else [])


_proj_outs = (jax.ShapeDtypeStruct((NPAD, D), _f32),
              jax.ShapeDtypeStruct((NPAD, D), _f32),
              jax.ShapeDtypeStruct((NPAD, 16), _f32),
              jax.ShapeDtypeStruct((NPAD, 16), _f32))


def kernel(x_, edge_index, W_emb, b_emb, W_gat, att_src, att_dst, b_gat,
           W_ff1, b_ff1, W_ff2, b_ff2, bn_gamma, bn_beta):
    loop = jnp.arange(N, dtype=jnp.int32)
    padE = jnp.full((EPAD - E - N,), N, jnp.int32)
    sd = jnp.stack([
        jnp.concatenate([edge_index[0], loop, padE]),
        jnp.concatenate([edge_index[1], loop, padE]),
    ]).reshape(2, NC * NS, NCHUNK, CHUNK).transpose(1, 2, 0, 3)
    idx3 = jnp.concatenate(
        [sd, jnp.full((NC * NS, 2, 2, CHUNK), N, jnp.int32)], axis=1)
    xp = jnp.zeros((NPAD, 8), _f32).at[:N, :3].set(x_)
    wemb = jnp.zeros((8, D), _f32).at[:3].set(W_emb)
    zA = jnp.zeros((NPAD, D), _f32)
    zD = jnp.zeros((NPAD, 16), _f32)
    r1 = lambda a: a.reshape(1, -1)
    afs = [att_src[i].reshape(1, D) for i in range(L)]
    afd = [att_dst[i].reshape(1, D) for i in range(L)]

    x, h, as_, ad_ = _tc_call(_tc_embed_a, _proj_outs)(
        xp, wemb, r1(b_emb), W_gat[0], afs[0], afd[0])

    for i in range(L):
        accA, accD = _sc_edge(h, as_, ad_, idx3, zA, zD)
        bargs = (x, accA, accD, r1(b_gat[i]),
                 r1(bn_gamma[2 * i]), r1(bn_beta[2 * i]),
                 r1(bn_gamma[2 * i + 1]), r1(bn_beta[2 * i + 1]),
                 W_ff1[i], r1(b_ff1[i]), W_ff2[i], r1(b_ff2[i]))
        x = _tc_call(_tc_b, jax.ShapeDtypeStruct((NPAD, D), _f32),
                     scratch=True)(*bargs)
        if i < L - 1:
            h, as_, ad_ = _tc_call(_tc_a, _proj_outs[1:])(
                x, W_gat[i + 1], afs[i + 1], afd[i + 1])
    return x[:N]


# speedup vs baseline: 1.4731x; 1.4731x over previous
import functools
import jax
import jax.numpy as jnp
from jax import lax
from jax.experimental import pallas as pl
from jax.experimental.pallas import tpu as pltpu
from jax.experimental.pallas import tpu_sc as plsc

N = 10000
D = 128
H = 8
C = 16
L = 3
FF = 512
NPAD = 10112
E = 320000
EPAD = 331776
NC, NS = 2, 16
PER_TILE = EPAD // (NC * NS)
CHUNK = 128
NCHUNK = PER_TILE // CHUNK
STRIPE = NPAD // NS

_mm = functools.partial(jnp.dot, precision=lax.Precision.HIGHEST)
_f32 = jnp.float32


def _rowmask():
    return lax.broadcasted_iota(jnp.int32, (NPAD, D), 0) < N


def _headsel():
    r = lax.broadcasted_iota(jnp.int32, (D, 16), 0) // 16
    c = lax.broadcasted_iota(jnp.int32, (D, 16), 1)
    return (r == c).astype(_f32)


def _headexp():
    r = lax.broadcasted_iota(jnp.int32, (16, D), 0)
    c = lax.broadcasted_iota(jnp.int32, (16, D), 1) // 16
    return (r == c).astype(_f32)


def _halfsum():
    r = lax.broadcasted_iota(jnp.int32, (32, 16), 0) % 16
    c = lax.broadcasted_iota(jnp.int32, (32, 16), 1)
    return (r == c).astype(_f32)


def _proj(x, wgat, af_s, af_d, h_o, as_o, ad_o):
    h = _mm(x, wgat)
    S = _headsel()
    h_o[...] = h
    as_o[...] = _mm(h * af_s, S)
    ad_o[...] = _mm(h * af_d, S)


def _tc_embed_a(xp, wemb, bemb, wgat, af_s, af_d,
                x_o, h_o, as_o, ad_o):
    x = jnp.where(_rowmask(), _mm(xp[...], wemb[...]) + bemb[...], 0.0)
    x_o[...] = x
    _proj(x, wgat[...], af_s[...], af_d[...], h_o, as_o, ad_o)


_FFBLK = 632


def _postgat(x, accA, accD, bgat, g1, b1, g2, b2, w1, bf1, w2, bf2, u_ref,
             v_ref):
    mask = _rowmask()
    inv = 1.0 / (_mm(accD, _halfsum()) + 1e-16)
    y = (accA[0] + accA[1]) * _mm(inv, _headexp()) + bgat
    t = x + y
    mu = jnp.sum(t, axis=0, keepdims=True) / N
    var = jnp.sum(t * t, axis=0, keepdims=True) / N - mu * mu
    u_ref[...] = jnp.where(mask, g1 * (t - mu) * lax.rsqrt(var + 1e-5) + b1,
                           0.0)

    def blk(i, carry):
        off = pl.multiple_of(i * _FFBLK, 8)
        ub = u_ref[pl.ds(off, _FFBLK), :]
        fb = jax.nn.relu(_mm(ub, w1) + bf1)
        v_ref[pl.ds(off, _FFBLK), :] = ub + _mm(fb, w2) + bf2
        return carry

    lax.fori_loop(0, NPAD // _FFBLK, blk, 0)
    v = v_ref[...]
    mu2 = jnp.sum(v, axis=0, keepdims=True) / N
    var2 = jnp.sum(v * v, axis=0, keepdims=True) / N - mu2 * mu2
    return jnp.where(mask, g2 * (v - mu2) * lax.rsqrt(var2 + 1e-5) + b2, 0.0)


def _tc_b(x, accA, accD, bgat, g1, b1, g2, b2, w1, bf1, w2, bf2, x_o,
          u_ref, v_ref):
    x_o[...] = _postgat(x[...], accA[...], accD[...], bgat[...], g1[...],
                        b1[...], g2[...], b2[...], w1[...], bf1[...],
                        w2[...], bf2[...], u_ref, v_ref)


def _tc_a(x, wgat, af_s, af_d, h_o, as_o, ad_o):
    _proj(x[...], wgat[...], af_s[...], af_d[...], h_o, as_o, ad_o)


def _bcast_lane(vec, h):
    idx = jnp.full((16, 1), h, jnp.int32)
    dn = lax.GatherDimensionNumbers(offset_dims=(), collapsed_slice_dims=(0,),
                                    start_index_map=(0,))
    return lax.gather(vec, idx, dn, (1,),
                      mode=lax.GatherScatterMode.PROMISE_IN_BOUNDS)


def _sc_edge_body(h_hbm, as_hbm, ad_hbm, src_hbm, dst_hbm, zA_hbm, zD_hbm,
                  accA_o, accD_o,
                  idx_v, hrow_v, as_v, ad_v, w_v, msg_v, accA_s, accD_s,
                  sem1, sem2, sem3):
    cid = lax.axis_index("c")
    sid = lax.axis_index("s")
    wid = sid * NC + cid
    base_rows = sid * STRIPE
    pltpu.sync_copy(zA_hbm.at[pl.ds(base_rows, STRIPE)],
                    accA_s.at[pl.ds(base_rows, STRIPE)])
    pltpu.sync_copy(zD_hbm.at[pl.ds(base_rows, STRIPE)],
                    accD_s.at[pl.ds(base_rows, STRIPE)])
    plsc.subcore_barrier()

    mask16 = jnp.where(lax.iota(jnp.int32, 16) < 8, 1.0, 0.0).astype(_f32)

    def chunk_body(ci, carry):
        base = wid * PER_TILE + ci * CHUNK
        pltpu.sync_copy(src_hbm.at[pl.ds(base, CHUNK)], idx_v.at[0])
        pltpu.sync_copy(dst_hbm.at[pl.ds(base, CHUNK)], idx_v.at[1])
        cp1 = pltpu.async_copy(h_hbm.at[idx_v.at[0]], hrow_v, sem1)
        cp2 = pltpu.async_copy(as_hbm.at[idx_v.at[0]], as_v, sem2)
        cp3 = pltpu.async_copy(ad_hbm.at[idx_v.at[1]], ad_v, sem3)
        cp2.wait()
        cp3.wait()

        def edge_w(e, c2):
            a = as_v[e, :] + ad_v[e, :]
            w_v[e, :] = jnp.exp(jnp.maximum(a, 0.2 * a)) * mask16
            return c2

        lax.fori_loop(0, CHUNK, edge_w, 0)
        cp1.wait()

        def edge_m(e, c2):
            wrow = w_v[e, :]
            for h in range(H):
                mv = hrow_v[e, pl.ds(h * 16, 16)] * _bcast_lane(wrow, h)
                msg_v[e, pl.ds(h * 16, 16)] = mv
            return c2

        lax.fori_loop(0, CHUNK, edge_m, 0)
        pltpu.sync_copy(w_v, accD_s.at[idx_v.at[1]], add=True)
        pltpu.sync_copy(msg_v, accA_s.at[idx_v.at[1]], add=True)
        return carry

    lax.fori_loop(0, NCHUNK, chunk_body, 0)
    plsc.subcore_barrier()
    pltpu.sync_copy(accA_s.at[pl.ds(base_rows, STRIPE)],
                    accA_o.at[cid, pl.ds(base_rows, STRIPE)])
    pltpu.sync_copy(accD_s.at[pl.ds(base_rows, STRIPE)],
                    accD_o.at[pl.ds(base_rows, STRIPE), pl.ds(cid * 16, 16)])


_sc_edge = pl.kernel(
    _sc_edge_body,
    out_type=(jax.ShapeDtypeStruct((NC, NPAD, D), _f32),
              jax.ShapeDtypeStruct((NPAD, 32), _f32)),
    mesh=plsc.VectorSubcoreMesh(core_axis_name="c", subcore_axis_name="s"),
    compiler_params=pltpu.CompilerParams(use_tc_tiling_on_sc=False),
    scratch_types=(
        pltpu.VMEM((2, CHUNK), jnp.int32),
        pltpu.VMEM((CHUNK, D), _f32),
        pltpu.VMEM((CHUNK, 16), _f32),
        pltpu.VMEM((CHUNK, 16), _f32),
        pltpu.VMEM((CHUNK, 16), _f32),
        pltpu.VMEM((CHUNK, D), _f32),
        pltpu.VMEM_SHARED((NPAD, D), _f32),
        pltpu.VMEM_SHARED((NPAD, 16), _f32),
        pltpu.SemaphoreType.DMA,
        pltpu.SemaphoreType.DMA,
        pltpu.SemaphoreType.DMA,
    ),
)


def _tc_call(body, n_out_like, scratch=False):
    return pl.pallas_call(
        body, out_shape=n_out_like,
        scratch_shapes=[pltpu.VMEM((NPAD, D), _f32),
                        pltpu.VMEM((NPAD, D), _f32)] if scratch else [])


_proj_outs = (jax.ShapeDtypeStruct((NPAD, D), _f32),
              jax.ShapeDtypeStruct((NPAD, D), _f32),
              jax.ShapeDtypeStruct((NPAD, 16), _f32),
              jax.ShapeDtypeStruct((NPAD, 16), _f32))


def kernel(x_, edge_index, W_emb, b_emb, W_gat, att_src, att_dst, b_gat,
           W_ff1, b_ff1, W_ff2, b_ff2, bn_gamma, bn_beta):
    loop = jnp.arange(N, dtype=jnp.int32)
    padE = jnp.full((EPAD - E - N,), N, jnp.int32)
    src = jnp.concatenate([edge_index[0], loop, padE])
    dst = jnp.concatenate([edge_index[1], loop, padE])
    xp = jnp.zeros((NPAD, 8), _f32).at[:N, :3].set(x_)
    wemb = jnp.zeros((8, D), _f32).at[:3].set(W_emb)
    zA = jnp.zeros((NPAD, D), _f32)
    zD = jnp.zeros((NPAD, 16), _f32)
    r1 = lambda a: a.reshape(1, -1)
    afs = [att_src[i].reshape(1, D) for i in range(L)]
    afd = [att_dst[i].reshape(1, D) for i in range(L)]

    x, h, as_, ad_ = _tc_call(_tc_embed_a, _proj_outs)(
        xp, wemb, r1(b_emb), W_gat[0], afs[0], afd[0])

    for i in range(L):
        accA, accD = _sc_edge(h, as_, ad_, src, dst, zA, zD)
        bargs = (x, accA, accD, r1(b_gat[i]),
                 r1(bn_gamma[2 * i]), r1(bn_beta[2 * i]),
                 r1(bn_gamma[2 * i + 1]), r1(bn_beta[2 * i + 1]),
                 W_ff1[i], r1(b_ff1[i]), W_ff2[i], r1(b_ff2[i]))
        x = _tc_call(_tc_b, jax.ShapeDtypeStruct((NPAD, D), _f32),
                     scratch=True)(*bargs)
        if i < L - 1:
            h, as_, ad_ = _tc_call(_tc_a, _proj_outs[1:])(
                x, W_gat[i + 1], afs[i + 1], afd[i + 1])
    return x[:N]
